# Initial kernel scaffold; baseline (speedup 1.0000x reference)
import jax
import jax.numpy as jnp
from jax.experimental import pallas as pl


def kernel(v1, v2, y, idx, memory_v1, memory_v2):
    raise NotImplementedError("write your pallas kernel here")


# speedup vs baseline: 3.1804x; 3.1804x over previous
import dataclasses
import functools
import math

import jax
import jax.numpy as jnp
from jax import lax
from jax.experimental import pallas as pl
from jax.experimental.pallas import tpu as pltpu
from jax.experimental.pallas import tpu_sc as plsc

B = 1024
D = 128
K = 1025
OUT = 100000
T = 0.07
MOM = 0.5

KP = 1056
C = 96
NCH = KP // C
NC = 2
NS = 16
NW = NC * NS
BPW = B // NW
L = 16


def _dot_chunk(rows_ref, vslices, dst_ref, lane):
    @pl.loop(0, C // L)
    def _(g):
        dvec = jnp.zeros((L,), jnp.float32)
        for rr in range(L):
            r = g * L + rr
            acc = rows_ref[r, pl.ds(0, L)] * vslices[0]
            for t in range(1, D // L):
                acc = acc + rows_ref[r, pl.ds(t * L, L)] * vslices[t]
            dvec = jnp.where(lane == rr, jnp.sum(acc), dvec)
        dst_ref[pl.ds(g * L, L)] = dvec


def _sc_gather_dots(mem1r, mem2r, mem1, mem2, idx_flat, y, v1, v2):
    mesh = plsc.VectorSubcoreMesh(core_axis_name="c", subcore_axis_name="s")
    cp = pltpu.CompilerParams()
    if "needs_layout_passes" in pltpu.CompilerParams.__dataclass_fields__:
        cp = dataclasses.replace(cp, needs_layout_passes=False)

    @functools.partial(
        pl.kernel,
        mesh=mesh,
        compiler_params=cp,
        out_type=[
            jax.ShapeDtypeStruct((B * KP,), jnp.float32),
            jax.ShapeDtypeStruct((B * KP,), jnp.float32),
            jax.ShapeDtypeStruct((B, D), jnp.float32),
            jax.ShapeDtypeStruct((B, D), jnp.float32),
        ],
        scratch_types=[
            pltpu.VMEM((C,), jnp.int32),
            pltpu.VMEM((C, D), jnp.float32),
            pltpu.VMEM((C, D), jnp.float32),
            pltpu.VMEM((D,), jnp.float32),
            pltpu.VMEM((D,), jnp.float32),
            pltpu.VMEM((C,), jnp.float32),
            pltpu.VMEM((C,), jnp.float32),
            pltpu.VMEM((BPW,), jnp.int32),
            pltpu.VMEM((BPW, D), jnp.float32),
            pltpu.SemaphoreType.DMA,
            pltpu.SemaphoreType.DMA,
        ],
    )
    def k(mem1r_hbm, mem2r_hbm, mem1_hbm, mem2_hbm, idx_hbm, y_hbm,
          v1_hbm, v2_hbm,
          dots1_hbm, dots2_hbm, pos1_hbm, pos2_hbm,
          idx_v, rows1_v, rows2_v, v1row_v, v2row_v, d1_v, d2_v,
          y_v, pos_v, sem1, sem2):
        wid = lax.axis_index("s") * NC + lax.axis_index("c")
        lane = lax.iota(jnp.int32, L)
        wbase = wid * BPW

        pltpu.sync_copy(y_hbm.at[pl.ds(wbase, BPW)], y_v)
        pltpu.async_copy(mem1_hbm.at[y_v], pos_v, sem1).wait()
        pltpu.sync_copy(pos_v, pos1_hbm.at[pl.ds(wbase, BPW)])
        pltpu.async_copy(mem2_hbm.at[y_v], pos_v, sem1).wait()
        pltpu.sync_copy(pos_v, pos2_hbm.at[pl.ds(wbase, BPW)])

        @pl.loop(0, BPW)
        def _(bb):
            b = wbase + bb
            pltpu.sync_copy(v1_hbm.at[b], v1row_v)
            pltpu.sync_copy(v2_hbm.at[b], v2row_v)
            v1s = [v1row_v[pl.ds(t * L, L)] for t in range(D // L)]
            v2s = [v2row_v[pl.ds(t * L, L)] for t in range(D // L)]

            @pl.loop(0, NCH)
            def _(ch):
                base = b * KP + ch * C
                pltpu.sync_copy(idx_hbm.at[pl.ds(base, C)], idx_v)
                cp1 = pltpu.async_copy(mem1r_hbm.at[idx_v], rows1_v, sem1)
                cp2 = pltpu.async_copy(mem2r_hbm.at[idx_v], rows2_v, sem2)
                cp1.wait()
                cp2.wait()

                _dot_chunk(rows1_v, v2s, d1_v, lane)
                _dot_chunk(rows2_v, v1s, d2_v, lane)
                pltpu.sync_copy(d1_v, dots1_hbm.at[pl.ds(base, C)])
                pltpu.sync_copy(d2_v, dots2_hbm.at[pl.ds(base, C)])

    return k(mem1r, mem2r, mem1, mem2, idx_flat, y, v1, v2)


def _tc_merge(dots1, dots2, pos1, pos2, v1, v2):
    def body(d1_ref, d2_ref, p1_ref, p2_ref, v1_ref, v2_ref,
             o2_ref, o1_ref, u1_ref, u2_ref):
        col = lax.broadcasted_iota(jnp.int32, (B, KP), 1)
        mask = col < K
        e1 = jnp.where(mask, jnp.exp(d1_ref[...] / T), 0.0)
        e2 = jnp.where(mask, jnp.exp(d2_ref[...] / T), 0.0)
        z_v2 = jnp.sum(e1) / (B * K) * OUT
        z_v1 = jnp.sum(e2) / (B * K) * OUT
        o2_ref[...] = e1 / z_v2
        o1_ref[...] = e2 / z_v1

        l1 = p1_ref[...] * MOM + v1_ref[...] * (1.0 - MOM)
        n1 = jnp.sqrt(jnp.sum(l1 * l1, axis=1, keepdims=True))
        u1_ref[...] = l1 / n1
        l2 = p2_ref[...] * MOM + v2_ref[...] * (1.0 - MOM)
        n2 = jnp.sqrt(jnp.sum(l2 * l2, axis=1, keepdims=True))
        u2_ref[...] = l2 / n2

    return pl.pallas_call(
        body,
        out_shape=[
            jax.ShapeDtypeStruct((B, KP), jnp.float32),
            jax.ShapeDtypeStruct((B, KP), jnp.float32),
            jax.ShapeDtypeStruct((B, D), jnp.float32),
            jax.ShapeDtypeStruct((B, D), jnp.float32),
        ],
    )(dots1, dots2, pos1, pos2, v1, v2)


def _tc_scatter(y, bank, upd):
    def body(y_ref, bank_ref, upd_ref, out_ref):
        out_ref[...] = upd_ref[...]

    out = pl.pallas_call(
        body,
        grid_spec=pltpu.PrefetchScalarGridSpec(
            num_scalar_prefetch=1,
            grid=(B,),
            in_specs=[
                pl.BlockSpec(memory_space=pl.ANY),
                pl.BlockSpec((1, 1, D), lambda i, y_ref: (i, 0, 0)),
            ],
            out_specs=pl.BlockSpec((1, 1, D), lambda i, y_ref: (y_ref[i], 0, 0)),
        ),
        out_shape=jax.ShapeDtypeStruct((OUT, 1, D), jnp.float32),
        input_output_aliases={1: 0},
    )(y, bank.reshape(OUT, 1, D), upd.reshape(B, 1, D))
    return out.reshape(OUT, D)


@jax.jit
def kernel(v1, v2, y, idx, memory_v1, memory_v2):
    y = y.astype(jnp.int32)
    idx_full = jnp.concatenate([y[:, None], idx[:, 1:].astype(jnp.int32)], axis=1)
    idx_pad = jnp.pad(idx_full, ((0, 0), (0, KP - K)))
    idx_flat = idx_pad.reshape(-1)

    rr = lambda x: x.astype(jnp.bfloat16).astype(jnp.float32)
    dots1, dots2, pos1, pos2 = _sc_gather_dots(
        rr(memory_v1), rr(memory_v2), memory_v1, memory_v2,
        idx_flat, y, rr(v1), rr(v2))
    dots1 = dots1.reshape(B, KP)
    dots2 = dots2.reshape(B, KP)

    o_v2, o_v1, u1, u2 = _tc_merge(dots1, dots2, pos1, pos2, v1, v2)

    new_mem1 = _tc_scatter(y, memory_v1, u1)
    new_mem2 = _tc_scatter(y, memory_v2, u2)

    out_v1 = o_v1[:, :K, None]
    out_v2 = o_v2[:, :K, None]
    return (out_v1, out_v2, new_mem1, new_mem2)


# speedup vs baseline: 4.5661x; 1.4357x over previous
import dataclasses
import functools
import math

import jax
import jax.numpy as jnp
from jax import lax
from jax.experimental import pallas as pl
from jax.experimental.pallas import tpu as pltpu
from jax.experimental.pallas import tpu_sc as plsc

B = 1024
D = 128
K = 1025
OUT = 100000
T = 0.07
MOM = 0.5

KP = 1056
C = 96
NCH = KP // C
NC = 2
NS = 16
NW = NC * NS
BPW = B // NW
L = 16
NT = BPW * NCH


def _dot_chunk(rows_ref, vslices, dst_ref, lane):
    @pl.loop(0, C // L)
    def _(g):
        dvec = jnp.zeros((L,), jnp.float32)
        for rr in range(L):
            r = g * L + rr
            acc = rows_ref[r, pl.ds(0, L)] * vslices[0]
            for t in range(1, D // L):
                acc = acc + rows_ref[r, pl.ds(t * L, L)] * vslices[t]
            dvec = jnp.where(lane == rr, jnp.sum(acc), dvec)
        dst_ref[pl.ds(g * L, L)] = dvec


def _sc_gather_dots(mem1r, mem2r, mem1, mem2, idx_flat, y, v1, v2):
    mesh = plsc.VectorSubcoreMesh(core_axis_name="c", subcore_axis_name="s")
    cp = pltpu.CompilerParams()
    if "needs_layout_passes" in pltpu.CompilerParams.__dataclass_fields__:
        cp = dataclasses.replace(cp, needs_layout_passes=False)

    @functools.partial(
        pl.kernel,
        mesh=mesh,
        compiler_params=cp,
        out_type=[
            jax.ShapeDtypeStruct((B * KP,), jnp.float32),
            jax.ShapeDtypeStruct((B * KP,), jnp.float32),
            jax.ShapeDtypeStruct((B, D), jnp.float32),
            jax.ShapeDtypeStruct((B, D), jnp.float32),
        ],
        scratch_types=[
            pltpu.VMEM((NT * C,), jnp.int32),
            pltpu.VMEM((C, D), jnp.float32),
            pltpu.VMEM((C, D), jnp.float32),
            pltpu.VMEM((C, D), jnp.float32),
            pltpu.VMEM((C, D), jnp.float32),
            pltpu.VMEM((BPW, D), jnp.float32),
            pltpu.VMEM((BPW, D), jnp.float32),
            pltpu.VMEM((C,), jnp.float32),
            pltpu.VMEM((C,), jnp.float32),
            pltpu.VMEM((C,), jnp.float32),
            pltpu.VMEM((C,), jnp.float32),
            pltpu.VMEM((BPW,), jnp.int32),
            pltpu.VMEM((BPW, D), jnp.float32),
            pltpu.SemaphoreType.DMA,
            pltpu.SemaphoreType.DMA,
            pltpu.SemaphoreType.DMA,
            pltpu.SemaphoreType.DMA,
            pltpu.SemaphoreType.DMA,
            pltpu.SemaphoreType.DMA,
        ],
    )
    def k(mem1r_hbm, mem2r_hbm, mem1_hbm, mem2_hbm, idx_hbm, y_hbm,
          v1_hbm, v2_hbm,
          dots1_hbm, dots2_hbm, pos1_hbm, pos2_hbm,
          idxw_v, r1a_v, r2a_v, r1b_v, r2b_v, v1w_v, v2w_v,
          d1a_v, d2a_v, d1b_v, d2b_v, y_v, pos_v,
          sa1, sa2, sb1, sb2, dsema, dsemb):
        wid = lax.axis_index("s") * NC + lax.axis_index("c")
        lane = lax.iota(jnp.int32, L)
        wbase = wid * BPW
        dots_base = wbase * KP

        pltpu.sync_copy(y_hbm.at[pl.ds(wbase, BPW)], y_v)
        pltpu.async_copy(mem1_hbm.at[y_v], pos_v, sa1).wait()
        pltpu.sync_copy(pos_v, pos1_hbm.at[pl.ds(wbase, BPW)])
        pltpu.async_copy(mem2_hbm.at[y_v], pos_v, sa1).wait()
        pltpu.sync_copy(pos_v, pos2_hbm.at[pl.ds(wbase, BPW)])

        pltpu.sync_copy(idx_hbm.at[pl.ds(wbase * KP, NT * C)], idxw_v)
        pltpu.sync_copy(v1_hbm.at[pl.ds(wbase, BPW)], v1w_v)
        pltpu.sync_copy(v2_hbm.at[pl.ds(wbase, BPW)], v2w_v)

        def start(t, r1, r2, s1, s2):
            src = idxw_v.at[pl.ds(t * C, C)]
            pltpu.async_copy(mem1r_hbm.at[src], r1, s1)
            pltpu.async_copy(mem2r_hbm.at[src], r2, s2)

        def wait_gathers(r1, r2, s1, s2):
            src = idxw_v.at[pl.ds(0, C)]
            pltpu.make_async_copy(mem1r_hbm.at[src], r1, s1).wait()
            pltpu.make_async_copy(mem2r_hbm.at[src], r2, s2).wait()

        def wait_emits(d1, d2, ds_):
            pltpu.make_async_copy(d1, dots1_hbm.at[pl.ds(dots_base, C)], ds_).wait()
            pltpu.make_async_copy(d2, dots2_hbm.at[pl.ds(dots_base, C)], ds_).wait()

        def compute_emit(t, r1, r2, d1, d2, ds_):
            bb = t // NCH
            v1s = [v1w_v[bb, pl.ds(u * L, L)] for u in range(D // L)]
            v2s = [v2w_v[bb, pl.ds(u * L, L)] for u in range(D // L)]
            _dot_chunk(r1, v2s, d1, lane)
            _dot_chunk(r2, v1s, d2, lane)
            pltpu.async_copy(d1, dots1_hbm.at[pl.ds(dots_base + t * C, C)], ds_)
            pltpu.async_copy(d2, dots2_hbm.at[pl.ds(dots_base + t * C, C)], ds_)

        start(0, r1a_v, r2a_v, sa1, sa2)

        @pl.loop(0, NT // 2)
        def _(tt):
            t0 = 2 * tt

            start(t0 + 1, r1b_v, r2b_v, sb1, sb2)
            wait_gathers(r1a_v, r2a_v, sa1, sa2)

            @pl.when(tt > 0)
            def _():
                wait_emits(d1a_v, d2a_v, dsema)

            compute_emit(t0, r1a_v, r2a_v, d1a_v, d2a_v, dsema)

            @pl.when(t0 + 2 < NT)
            def _():
                start(t0 + 2, r1a_v, r2a_v, sa1, sa2)

            wait_gathers(r1b_v, r2b_v, sb1, sb2)

            @pl.when(tt > 0)
            def _():
                wait_emits(d1b_v, d2b_v, dsemb)

            compute_emit(t0 + 1, r1b_v, r2b_v, d1b_v, d2b_v, dsemb)

        wait_emits(d1a_v, d2a_v, dsema)
        wait_emits(d1b_v, d2b_v, dsemb)

    return k(mem1r, mem2r, mem1, mem2, idx_flat, y, v1, v2)


def _tc_round_banks(mem1, mem2):
    RB = 1000

    def body(m1_ref, m2_ref, o1_ref, o2_ref):
        o1_ref[...] = m1_ref[...].astype(jnp.bfloat16).astype(jnp.float32)
        o2_ref[...] = m2_ref[...].astype(jnp.bfloat16).astype(jnp.float32)

    return pl.pallas_call(
        body,
        grid=(OUT // RB,),
        in_specs=[pl.BlockSpec((RB, D), lambda i: (i, 0)),
                  pl.BlockSpec((RB, D), lambda i: (i, 0))],
        out_specs=[pl.BlockSpec((RB, D), lambda i: (i, 0)),
                   pl.BlockSpec((RB, D), lambda i: (i, 0))],
        out_shape=[jax.ShapeDtypeStruct((OUT, D), jnp.float32),
                   jax.ShapeDtypeStruct((OUT, D), jnp.float32)],
    )(mem1, mem2)


def _tc_round_v(v1, v2):
    def body(a_ref, b_ref, oa_ref, ob_ref):
        oa_ref[...] = a_ref[...].astype(jnp.bfloat16).astype(jnp.float32)
        ob_ref[...] = b_ref[...].astype(jnp.bfloat16).astype(jnp.float32)

    return pl.pallas_call(
        body,
        out_shape=[jax.ShapeDtypeStruct((B, D), jnp.float32),
                   jax.ShapeDtypeStruct((B, D), jnp.float32)],
    )(v1, v2)


def _tc_merge(dots1, dots2, pos1, pos2, v1, v2):
    def body(d1_ref, d2_ref, p1_ref, p2_ref, v1_ref, v2_ref,
             o2_ref, o1_ref, u1_ref, u2_ref):
        col = lax.broadcasted_iota(jnp.int32, (B, KP), 1)
        mask = col < K
        e1 = jnp.where(mask, jnp.exp(d1_ref[...] / T), 0.0)
        e2 = jnp.where(mask, jnp.exp(d2_ref[...] / T), 0.0)
        z_v2 = jnp.sum(e1) / (B * K) * OUT
        z_v1 = jnp.sum(e2) / (B * K) * OUT
        o2_ref[...] = e1 / z_v2
        o1_ref[...] = e2 / z_v1

        l1 = p1_ref[...] * MOM + v1_ref[...] * (1.0 - MOM)
        n1 = jnp.sqrt(jnp.sum(l1 * l1, axis=1, keepdims=True))
        u1_ref[...] = l1 / n1
        l2 = p2_ref[...] * MOM + v2_ref[...] * (1.0 - MOM)
        n2 = jnp.sqrt(jnp.sum(l2 * l2, axis=1, keepdims=True))
        u2_ref[...] = l2 / n2

    return pl.pallas_call(
        body,
        out_shape=[
            jax.ShapeDtypeStruct((B, KP), jnp.float32),
            jax.ShapeDtypeStruct((B, KP), jnp.float32),
            jax.ShapeDtypeStruct((B, D), jnp.float32),
            jax.ShapeDtypeStruct((B, D), jnp.float32),
        ],
    )(dots1, dots2, pos1, pos2, v1, v2)


def _tc_scatter2(y, winner, bank1, bank2, upd1, upd2):
    def body(y_ref, w_ref, u1_ref, u2_ref, b1_ref, b2_ref,
             o1_ref, o2_ref, sem):
        def issue(i, carry):
            yi = y_ref[i]
            wi = w_ref[i]
            pltpu.make_async_copy(u1_ref.at[wi], o1_ref.at[yi], sem).start()
            pltpu.make_async_copy(u2_ref.at[wi], o2_ref.at[yi], sem).start()
            return carry

        lax.fori_loop(0, B, issue, 0)

        def drain(i, carry):
            pltpu.make_async_copy(u1_ref.at[0], o1_ref.at[0], sem).wait()
            pltpu.make_async_copy(u2_ref.at[0], o2_ref.at[0], sem).wait()
            return carry

        lax.fori_loop(0, B, drain, 0)

    return pl.pallas_call(
        body,
        in_specs=[
            pl.BlockSpec(memory_space=pltpu.SMEM),
            pl.BlockSpec(memory_space=pltpu.SMEM),
            pl.BlockSpec(memory_space=pltpu.VMEM),
            pl.BlockSpec(memory_space=pltpu.VMEM),
            pl.BlockSpec(memory_space=pl.ANY),
            pl.BlockSpec(memory_space=pl.ANY),
        ],
        out_specs=[
            pl.BlockSpec(memory_space=pl.ANY),
            pl.BlockSpec(memory_space=pl.ANY),
        ],
        out_shape=[
            jax.ShapeDtypeStruct((OUT, D), jnp.float32),
            jax.ShapeDtypeStruct((OUT, D), jnp.float32),
        ],
        input_output_aliases={4: 0, 5: 1},
        scratch_shapes=[pltpu.SemaphoreType.DMA],
    )(y, winner, upd1, upd2, bank1, bank2)


@jax.jit
def kernel(v1, v2, y, idx, memory_v1, memory_v2):
    y = y.astype(jnp.int32)
    idx_full = jnp.concatenate([y[:, None], idx[:, 1:].astype(jnp.int32)], axis=1)
    idx_pad = jnp.pad(idx_full, ((0, 0), (0, KP - K)))
    idx_flat = idx_pad.reshape(-1)

    mem1r, mem2r = _tc_round_banks(memory_v1, memory_v2)
    v1r, v2r = _tc_round_v(v1, v2)
    dots1, dots2, pos1, pos2 = _sc_gather_dots(
        mem1r, mem2r, memory_v1, memory_v2, idx_flat, y, v1r, v2r)
    dots1 = dots1.reshape(B, KP)
    dots2 = dots2.reshape(B, KP)

    o_v2, o_v1, u1, u2 = _tc_merge(dots1, dots2, pos1, pos2, v1, v2)

    jj = jnp.arange(B, dtype=jnp.int32)
    winner = jnp.max(jnp.where(y[:, None] == y[None, :], jj[None, :], -1),
                     axis=1).astype(jnp.int32)

    new_mem1, new_mem2 = _tc_scatter2(y, winner, memory_v1, memory_v2, u1, u2)

    out_v1 = o_v1[:, :K, None]
    out_v2 = o_v2[:, :K, None]
    return (out_v1, out_v2, new_mem1, new_mem2)
